# baseline (speedup 1.0000x reference)
import functools

import jax
import jax.numpy as jnp
from jax import lax
from jax.experimental import pallas as pl
from jax.experimental.pallas import tpu as pltpu
from jax.experimental.pallas import tpu_sc as plsc

_L = 16
_NSUB = 16
_K = 128


def _mm_body(x_ref, w_ref, b_ref, o_ref):
    o_ref[...] = (
        jnp.dot(x_ref[...], w_ref[0], preferred_element_type=jnp.float32)
        + b_ref[0]
    ).astype(jnp.bfloat16)


def _interleave_perm(H):
    perm = []
    for gblk in range(H // 32):
        for i in range(16):
            perm.append(gblk * 32 + i)
            perm.append(gblk * 32 + 16 + i)
    return perm


def _matmul_split(x, W, b):
    N, F = x.shape
    C = W.shape[1]
    H = C // 2
    RB = 2000
    nr = N // RB
    perm = jnp.asarray(_interleave_perm(H), jnp.int32)
    W2 = jnp.moveaxis(W.reshape(F, 2, H), 1, 0)[:, :, perm]
    b2 = b.reshape(2, 1, H)[:, :, perm]
    return pl.pallas_call(
        _mm_body,
        grid=(nr, 2),
        in_specs=[
            pl.BlockSpec((RB, F), lambda i, j: (i, 0)),
            pl.BlockSpec((1, F, H), lambda i, j: (j, 0, 0)),
            pl.BlockSpec((1, 1, H), lambda i, j: (j, 0, 0)),
        ],
        out_specs=pl.BlockSpec((RB, H), lambda i, j: (j * nr + i, 0)),
        out_shape=jax.ShapeDtypeStruct((2 * N, H), jnp.bfloat16),
    )(x, W2, b2)


def _splat_lane(v16, j):
    return lax.gather(
        v16,
        jnp.full((_L, 1), j, jnp.int32),
        lax.GatherDimensionNumbers(
            offset_dims=(),
            collapsed_slice_dims=(0,),
            start_index_map=(0,),
        ),
        slice_sizes=(1,),
        mode=lax.GatherScatterMode.PROMISE_IN_BOUNDS,
    )


def _make_sc_spmm(N, H, nchunks):
    RPT = -(-(-(-N // _NSUB)) // 128) * 128
    N_pad = RPT * _NSUB
    ZB = _K
    assert RPT % ZB == 0
    assert nchunks >= 6 and nchunks % 3 == 0
    ntrips = (nchunks - 3) // 3
    mesh = plsc.VectorSubcoreMesh(core_axis_name="c", subcore_axis_name="s")

    @functools.partial(
        pl.kernel,
        mesh=mesh,
        compiler_params=pltpu.CompilerParams(
            use_tc_tiling_on_sc=False, needs_layout_passes=False
        ),
        out_type=jax.ShapeDtypeStruct((2 * N_pad, H), jnp.float32),
        scratch_types=[
            pltpu.VMEM((nchunks, _K), jnp.int32),
            pltpu.VMEM((nchunks, _K), jnp.float32),
            pltpu.VMEM((_K,), jnp.int32),
            pltpu.VMEM((_K,), jnp.int32),
            pltpu.VMEM((_K,), jnp.int32),
            pltpu.VMEM((_K, H), jnp.bfloat16),
            pltpu.VMEM((_K, H), jnp.bfloat16),
            pltpu.VMEM((_K, H), jnp.bfloat16),
            pltpu.VMEM((_K, H), jnp.float32),
            pltpu.VMEM((_K, H), jnp.float32),
            pltpu.VMEM((_K, H), jnp.float32),
            pltpu.VMEM_SHARED((N_pad, H), jnp.float32),
            pltpu.SemaphoreType.DMA,
            pltpu.SemaphoreType.DMA,
            pltpu.SemaphoreType.DMA,
            pltpu.SemaphoreType.DMA,
            pltpu.SemaphoreType.DMA,
            pltpu.SemaphoreType.DMA,
        ],
    )
    def k(p_hbm, r_hbm, v_hbm, h_hbm, out_hbm,
          cb_all, vb_all, rb0, rb1, rb2, g0, g1, g2, o0, o1, o2, acc,
          gs0, gs1, gs2, ss0, ss1, ss2):
        c = lax.axis_index("c")
        s = lax.axis_index("s")
        rb = (rb0, rb1, rb2)
        g = (g0, g1, g2)
        o = (o0, o1, o2)
        gs = (gs0, gs1, gs2)
        ss = (ss0, ss1, ss2)

        pbase = (c * _NSUB + s) * nchunks
        vbase = s * nchunks
        idx_cp = pltpu.async_copy(
            p_hbm.at[pl.ds(pbase, nchunks)], cb_all, gs0
        )
        val_cp = pltpu.async_copy(v_hbm.at[pl.ds(vbase, nchunks)], vb_all, gs1)

        def _zrow(i, carry):
            for l in range(H // _L):
                o0[i, pl.ds(l * _L, _L)] = jnp.zeros((_L,), jnp.float32)
            return carry

        lax.fori_loop(0, ZB, _zrow, 0)
        r0 = s * RPT
        for i in range(RPT // ZB):
            pltpu.sync_copy(o0, acc.at[pl.ds(r0 + i * ZB, ZB)])
        idx_cp.wait()
        val_cp.wait()
        plsc.subcore_barrier()

        def start_gather(ki, b):
            pltpu.async_copy(r_hbm.at[vbase + ki], rb[b], gs[b])
            pltpu.async_copy(h_hbm.at[cb_all.at[ki]], g[b], gs[b])

        def wait_scatter(ki, b):
            pltpu.make_async_copy(o[b], acc.at[rb[b]], ss[b]).wait()

        def scale_and_scatter(ki, b):
            pltpu.make_async_copy(r_hbm.at[vbase + ki], rb[b], gs[b]).wait()
            pltpu.make_async_copy(h_hbm.at[cb_all.at[ki]], g[b], gs[b]).wait()

            for j16 in range(_K // _L):
                v16 = vb_all[ki, pl.ds(j16 * _L, _L)]
                for j in range(_L):
                    sv = _splat_lane(v16, j)
                    e = j16 * _L + j
                    for l in range(H // 32):
                        lo, hi = plsc.unpack(
                            g[b][e, pl.ds(l * 32, 32)],
                            format=plsc.PackFormat.INTERLEAVED,
                        )
                        o[b][e, pl.ds(l * 32, _L)] = lo * sv
                        o[b][e, pl.ds(l * 32 + _L, _L)] = hi * sv
            pltpu.async_copy(o[b], acc.at[rb[b]], ss[b], add=True)

        start_gather(0, 0)
        start_gather(1, 1)
        start_gather(2, 2)
        scale_and_scatter(0, 0)

        def _trip(t, carry):
            ki0 = 3 * t + 1
            for d in range(3):
                ki = ki0 + d
                b = (1 + d) % 3
                wait_scatter(ki - 1, (b + 2) % 3)
                start_gather(ki + 2, (b + 2) % 3)
                scale_and_scatter(ki, b)
            return carry

        lax.fori_loop(0, ntrips, _trip, 0)
        e2 = (nchunks - 2) % 3
        e1 = (nchunks - 1) % 3
        wait_scatter(nchunks - 3, (e2 + 2) % 3)
        scale_and_scatter(nchunks - 2, e2)
        wait_scatter(nchunks - 2, e2)
        scale_and_scatter(nchunks - 1, e1)
        wait_scatter(nchunks - 1, e1)
        plsc.subcore_barrier()

        for i in range(RPT // ZB):
            pltpu.sync_copy(acc.at[pl.ds(r0 + i * ZB, ZB)], o0)
            pltpu.sync_copy(
                o0, out_hbm.at[pl.ds(c * N_pad + r0 + i * ZB, ZB)]
            )

    return k


def kernel(x, S_indices, S_values, W, b):
    N, _ = x.shape
    C = W.shape[1]
    H = C // 2
    E = S_values.shape[0]

    nchunks = -(-(-(-E // _NSUB)) // _K)
    nchunks = max(6, -(-nchunks // 3) * 3)
    epw = nchunks * _K
    pad = epw * _NSUB - E
    rows_p = jnp.pad(S_indices[0], (0, pad)).reshape(_NSUB * nchunks, _K)
    cols_p = jnp.pad(S_indices[1], (0, pad)).reshape(_NSUB * nchunks, _K)
    vals_p = jnp.pad(S_values, (0, pad)).reshape(_NSUB * nchunks, _K)
    packed = jnp.concatenate([cols_p, cols_p + N], axis=0)

    h2 = _matmul_split(x, W, b)
    out2 = _make_sc_spmm(N, H, nchunks)(packed, rows_p, vals_p, h2)
    n_pad = out2.shape[0] // 2
    return jnp.concatenate([out2[:N], out2[n_pad:n_pad + N]], axis=1)


# speedup vs baseline: 1.1226x; 1.1226x over previous
import functools

import jax
import jax.numpy as jnp
from jax import lax
from jax.experimental import pallas as pl
from jax.experimental.pallas import tpu as pltpu
from jax.experimental.pallas import tpu_sc as plsc

_L = 16
_NSUB = 16
_K = 128


def _mm_body(x_ref, w_ref, b_ref, o_ref):
    o_ref[...] = (
        jnp.dot(x_ref[...], w_ref[0], preferred_element_type=jnp.float32)
        + b_ref[0]
    ).astype(jnp.bfloat16)


def _interleave_perm(H):
    perm = []
    for gblk in range(H // 32):
        for i in range(16):
            perm.append(gblk * 32 + i)
            perm.append(gblk * 32 + 16 + i)
    return perm


def _matmul_split(x, W, b):
    N, F = x.shape
    C = W.shape[1]
    H = C // 2
    RB = 2000
    nr = N // RB
    perm = jnp.asarray(_interleave_perm(H), jnp.int32)
    W2 = jnp.moveaxis(W.reshape(F, 2, H), 1, 0)[:, :, perm]
    b2 = b.reshape(2, 1, H)[:, :, perm]
    return pl.pallas_call(
        _mm_body,
        grid=(nr, 2),
        in_specs=[
            pl.BlockSpec((RB, F), lambda i, j: (i, 0)),
            pl.BlockSpec((1, F, H), lambda i, j: (j, 0, 0)),
            pl.BlockSpec((1, 1, H), lambda i, j: (j, 0, 0)),
        ],
        out_specs=pl.BlockSpec((RB, H), lambda i, j: (j * nr + i, 0)),
        out_shape=jax.ShapeDtypeStruct((2 * N, H), jnp.bfloat16),
    )(x, W2, b2)


def _splat_lane(v16, j):
    return lax.gather(
        v16,
        jnp.full((_L, 1), j, jnp.int32),
        lax.GatherDimensionNumbers(
            offset_dims=(),
            collapsed_slice_dims=(0,),
            start_index_map=(0,),
        ),
        slice_sizes=(1,),
        mode=lax.GatherScatterMode.PROMISE_IN_BOUNDS,
    )


def _make_sc_spmm(N, H, nchunks):
    RPT = -(-(-(-N // _NSUB)) // 128) * 128
    N_pad = RPT * _NSUB
    ZB = _K
    assert RPT % ZB == 0
    assert nchunks >= 4 and nchunks % 2 == 0
    npairs = (nchunks - 2) // 2
    mesh = plsc.VectorSubcoreMesh(core_axis_name="c", subcore_axis_name="s")

    @functools.partial(
        pl.kernel,
        mesh=mesh,
        compiler_params=pltpu.CompilerParams(
            use_tc_tiling_on_sc=False, needs_layout_passes=False
        ),
        out_type=jax.ShapeDtypeStruct((2 * N_pad, H), jnp.float32),
        scratch_types=[
            pltpu.VMEM((2 * nchunks, _K), jnp.int32),
            pltpu.VMEM((nchunks, _K), jnp.float32),
            pltpu.VMEM((_K, H), jnp.bfloat16),
            pltpu.VMEM((_K, H), jnp.bfloat16),
            pltpu.VMEM((_K, H), jnp.float32),
            pltpu.VMEM((_K, H), jnp.float32),
            pltpu.VMEM_SHARED((N_pad, H), jnp.float32),
            pltpu.SemaphoreType.DMA,
            pltpu.SemaphoreType.DMA,
            pltpu.SemaphoreType.DMA,
            pltpu.SemaphoreType.DMA,
        ],
    )
    def k(p_hbm, v_hbm, h_hbm, out_hbm,
          ib_all, vb_all, g0, g1, o0, o1, acc, gs0, gs1, ss0, ss1):
        c = lax.axis_index("c")
        s = lax.axis_index("s")
        g = (g0, g1)
        o = (o0, o1)
        gs = (gs0, gs1)
        ss = (ss0, ss1)

        pbase = (c * _NSUB + s) * nchunks
        vbase = s * nchunks
        idx_cp = pltpu.async_copy(
            p_hbm.at[pl.ds(2 * pbase, 2 * nchunks)], ib_all, gs0
        )
        val_cp = pltpu.async_copy(v_hbm.at[pl.ds(vbase, nchunks)], vb_all, gs1)

        def _zrow(i, carry):
            for l in range(H // _L):
                o0[i, pl.ds(l * _L, _L)] = jnp.zeros((_L,), jnp.float32)
            return carry

        lax.fori_loop(0, ZB, _zrow, 0)
        r0 = s * RPT
        for i in range(RPT // ZB):
            pltpu.sync_copy(o0, acc.at[pl.ds(r0 + i * ZB, ZB)])
        idx_cp.wait()
        val_cp.wait()
        plsc.subcore_barrier()

        def start_gather(ki, b):
            pltpu.async_copy(h_hbm.at[ib_all.at[2 * ki + 1]], g[b], gs[b])

        def wait_scatter(ki, b):
            pltpu.make_async_copy(
                o[b], acc.at[ib_all.at[2 * ki]], ss[b]
            ).wait()

        def scale_and_scatter(ki, b):
            pltpu.make_async_copy(
                h_hbm.at[ib_all.at[2 * ki + 1]], g[b], gs[b]
            ).wait()

            for j16 in range(_K // _L):
                v16 = vb_all[ki, pl.ds(j16 * _L, _L)]
                for j in range(_L):
                    sv = _splat_lane(v16, j)
                    e = j16 * _L + j
                    for l in range(H // 32):
                        lo, hi = plsc.unpack(
                            g[b][e, pl.ds(l * 32, 32)],
                            format=plsc.PackFormat.INTERLEAVED,
                        )
                        o[b][e, pl.ds(l * 32, _L)] = lo * sv
                        o[b][e, pl.ds(l * 32 + _L, _L)] = hi * sv
            pltpu.async_copy(o[b], acc.at[ib_all.at[2 * ki]], ss[b], add=True)

        start_gather(0, 0)
        start_gather(1, 1)
        scale_and_scatter(0, 0)

        def _pair(p, carry):
            ki = 2 * p + 1
            wait_scatter(ki - 1, 0)
            start_gather(ki + 1, 0)
            scale_and_scatter(ki, 1)
            wait_scatter(ki, 1)
            start_gather(ki + 2, 1)
            scale_and_scatter(ki + 1, 0)
            return carry

        lax.fori_loop(0, npairs, _pair, 0)
        scale_and_scatter(nchunks - 1, 1)
        wait_scatter(nchunks - 2, 0)
        wait_scatter(nchunks - 1, 1)
        plsc.subcore_barrier()

        for i in range(RPT // ZB):
            pltpu.sync_copy(acc.at[pl.ds(r0 + i * ZB, ZB)], o0)
            pltpu.sync_copy(
                o0, out_hbm.at[pl.ds(c * N_pad + r0 + i * ZB, ZB)]
            )

    return k


def kernel(x, S_indices, S_values, W, b):
    N, _ = x.shape
    C = W.shape[1]
    H = C // 2
    E = S_values.shape[0]

    nchunks = -(-(-(-E // _NSUB)) // _K)
    nchunks = max(4, nchunks + (nchunks % 2))
    epw = nchunks * _K
    pad = epw * _NSUB - E
    rows_p = jnp.pad(S_indices[0], (0, pad)).reshape(_NSUB * nchunks, _K)
    cols_p = jnp.pad(S_indices[1], (0, pad)).reshape(_NSUB * nchunks, _K)
    vals_p = jnp.pad(S_values, (0, pad)).reshape(_NSUB * nchunks, _K)
    packed = jnp.stack(
        [
            jnp.stack([rows_p, cols_p + c * N], axis=1)
            for c in range(2)
        ],
        axis=0,
    ).reshape(4 * _NSUB * nchunks, _K)

    h2 = _matmul_split(x, W, b)
    out2 = _make_sc_spmm(N, H, nchunks)(packed, vals_p, h2)
    n_pad = out2.shape[0] // 2
    return jnp.concatenate([out2[:N], out2[n_pad:n_pad + N]], axis=1)
